# baseline (speedup 1.0000x reference)
import functools

import jax
import jax.numpy as jnp
from jax import lax
from jax.experimental import pallas as pl
from jax.experimental.pallas import tpu as pltpu
from jax.experimental.pallas import tpu_sc as plsc

NC = 2
NS = 16
NW = NC * NS
B = 128
K = 4


def _worker_id():
    return lax.axis_index("s") * NC + lax.axis_index("c")


def _sc_mesh():
    return plsc.VectorSubcoreMesh(core_axis_name="c", subcore_axis_name="s")


_SC_PARAMS = pltpu.CompilerParams(use_tc_tiling_on_sc=False)


def _make_deg_kernel(NP, PG):
    @functools.partial(
        pl.kernel,
        mesh=_sc_mesh(),
        compiler_params=_SC_PARAMS,
        out_type=(jax.ShapeDtypeStruct((NP,), jnp.float32),
                  jax.ShapeDtypeStruct((NP,), jnp.float32)),
        scratch_types=[
            pltpu.VMEM((PG, B), jnp.int32),
            pltpu.VMEM((B,), jnp.float32),
            pltpu.VMEM_SHARED((NP,), jnp.float32),
            pltpu.SemaphoreType.DMA,
        ],
    )
    def deg_kernel(dst2d, zeros1, ones, out0, out1, dst_buf, ones_buf, acc,
                   ssem):
        c = lax.axis_index("c")
        s = lax.axis_index("s")
        w = _worker_id()

        @pl.when(s == 0)
        def _():
            pltpu.sync_copy(zeros1, acc)
        pltpu.sync_copy(ones, ones_buf)
        pltpu.sync_copy(dst2d.at[pl.ds(w * PG, PG)], dst_buf)
        plsc.subcore_barrier()

        def fire(j, _):
            pltpu.async_copy(ones_buf, acc.at[dst_buf.at[j]], ssem, add=True)
            return 0

        lax.fori_loop(0, PG, fire, 0)

        def drain(j, _):
            pltpu.make_async_copy(ones_buf, acc.at[dst_buf.at[j]], ssem).wait()
            return 0

        lax.fori_loop(0, PG, drain, 0)
        plsc.subcore_barrier()

        @pl.when((s == 0) & (c == 0))
        def _():
            pltpu.sync_copy(acc, out0)

        @pl.when((s == 0) & (c == 1))
        def _():
            pltpu.sync_copy(acc, out1)

    return deg_kernel


def _make_scatter0_kernel(NP, H, PG):
    RPT = NP // NS
    CH = PG // K
    assert PG % K == 0 and CH >= 4

    @functools.partial(
        pl.kernel,
        mesh=_sc_mesh(),
        compiler_params=_SC_PARAMS,
        out_type=(jax.ShapeDtypeStruct((NC, NP, 128), jnp.float32),
                  jax.ShapeDtypeStruct((NP,), jnp.float32),
                  jax.ShapeDtypeStruct((NP,), jnp.float32)),
        scratch_types=[
            pltpu.VMEM((PG, B), jnp.int32),
            pltpu.VMEM((PG, B), jnp.int32),
            pltpu.VMEM((2, K, B, H), jnp.float32),
            pltpu.VMEM((PG, B), jnp.float32),
            pltpu.VMEM_SHARED((NP, H), jnp.float32),
            pltpu.VMEM_SHARED((NP, H), jnp.float32),
            pltpu.VMEM_SHARED((NP,), jnp.float32),
            pltpu.VMEM_SHARED((NP,), jnp.float32),
            pltpu.SemaphoreType.DMA((2,)),
            pltpu.SemaphoreType.DMA((2,)),
            pltpu.SemaphoreType.DMA,
            pltpu.SemaphoreType.DMA,
        ],
    )
    def scatter0_kernel(g, src2d, dst2d, dinv, zeros1, out, s1o0, s1o1,
                        src_buf, dst_buf, rows, vals, acc, g_sh, acc1,
                        dinv_sh, gsem, ssem, gsem1, ssem1):
        c = lax.axis_index("c")
        s = lax.axis_index("s")
        w = _worker_id()

        pltpu.sync_copy(g.at[pl.ds(s * RPT, RPT), pl.ds(0, H)],
                        g_sh.at[pl.ds(s * RPT, RPT)])
        pltpu.sync_copy(dinv.at[pl.ds(s * RPT, RPT)],
                        dinv_sh.at[pl.ds(s * RPT, RPT)])

        @pl.when(s == 0)
        def _():
            pltpu.sync_copy(zeros1, acc1)
        zv = jnp.zeros((16,), jnp.float32)

        def zbody(i, _):
            r = i // (H // 16)
            k = i % (H // 16)
            rows[0, 0, r, pl.ds(k * 16, 16)] = zv
            return 0

        lax.fori_loop(0, B * (H // 16), zbody, 0)
        nfull = RPT // B
        for q in range(nfull):
            pltpu.sync_copy(rows.at[0, 0],
                            acc.at[pl.ds(s * RPT + q * B, B)])
        rem = RPT - nfull * B
        if rem:
            pltpu.sync_copy(rows.at[0, 0].at[pl.ds(0, rem)],
                            acc.at[pl.ds(s * RPT + nfull * B, rem)])
        pltpu.sync_copy(src2d.at[pl.ds(w * PG, PG)], src_buf)
        pltpu.sync_copy(dst2d.at[pl.ds(w * PG, PG)], dst_buf)
        plsc.subcore_barrier()

        def fire_gv(j, _):
            pltpu.async_copy(dinv_sh.at[dst_buf.at[j]], vals.at[j], gsem1)
            return 0

        lax.fori_loop(0, PG, fire_gv, 0)

        def fire_g(ch, p):
            for b in range(K):
                pltpu.async_copy(g_sh.at[src_buf.at[ch * K + b]],
                                 rows.at[p, b], gsem.at[p])

        def drain_g_fire_s(ch, p):
            for b in range(K):
                pltpu.make_async_copy(g_sh.at[src_buf.at[ch * K + b]],
                                      rows.at[p, b], gsem.at[p]).wait()
                pltpu.async_copy(rows.at[p, b],
                                 acc.at[dst_buf.at[ch * K + b]],
                                 ssem.at[p], add=True)

        def drain_s(ch, p):
            for b in range(K):
                pltpu.make_async_copy(rows.at[p, b],
                                      acc.at[dst_buf.at[ch * K + b]],
                                      ssem.at[p]).wait()

        fire_g(0, 0)
        fire_g(1, 1)
        drain_g_fire_s(0, 0)

        def body(ch, _):
            p = lax.rem(ch, 2)
            drain_s(ch - 2, p)
            fire_g(ch, p)
            drain_g_fire_s(ch - 1, 1 - p)
            return 0

        lax.fori_loop(2, CH, body, 0)
        drain_s(CH - 2, CH % 2)
        drain_g_fire_s(CH - 1, (CH - 1) % 2)
        drain_s(CH - 1, (CH - 1) % 2)

        def drain_gv(j, _):
            pltpu.make_async_copy(dinv_sh.at[dst_buf.at[j]], vals.at[j],
                                  gsem1).wait()
            return 0

        lax.fori_loop(0, PG, drain_gv, 0)

        def fire_s1(j, _):
            pltpu.async_copy(vals.at[j], acc1.at[src_buf.at[j]], ssem1,
                             add=True)
            return 0

        lax.fori_loop(0, PG, fire_s1, 0)

        def drain_s1(j, _):
            pltpu.make_async_copy(vals.at[j], acc1.at[src_buf.at[j]],
                                  ssem1).wait()
            return 0

        lax.fori_loop(0, PG, drain_s1, 0)
        plsc.subcore_barrier()

        pltpu.sync_copy(acc.at[pl.ds(s * RPT, RPT)],
                        out.at[c].at[pl.ds(s * RPT, RPT), pl.ds(0, H)])

        @pl.when((s == 0) & (c == 0))
        def _():
            pltpu.sync_copy(acc1, s1o0)

        @pl.when((s == 0) & (c == 1))
        def _():
            pltpu.sync_copy(acc1, s1o1)

    return scatter0_kernel


def _make_scatter_kernel(NP, H, PG):
    RPT = NP // NS
    CH = PG // K
    assert PG % K == 0 and CH >= 4

    @functools.partial(
        pl.kernel,
        mesh=_sc_mesh(),
        compiler_params=_SC_PARAMS,
        out_type=jax.ShapeDtypeStruct((NC, NP, 128), jnp.float32),
        scratch_types=[
            pltpu.VMEM((PG, B), jnp.int32),
            pltpu.VMEM((PG, B), jnp.int32),
            pltpu.VMEM((2, K, B, H), jnp.float32),
            pltpu.VMEM_SHARED((NP, H), jnp.float32),
            pltpu.VMEM_SHARED((NP, H), jnp.float32),
            pltpu.SemaphoreType.DMA((2,)),
            pltpu.SemaphoreType.DMA((2,)),
        ],
    )
    def scatter_kernel(g, src2d, dst2d, out, src_buf, dst_buf, rows,
                       acc, g_sh, gsem, ssem):
        c = lax.axis_index("c")
        s = lax.axis_index("s")
        w = _worker_id()

        pltpu.sync_copy(g.at[pl.ds(s * RPT, RPT), pl.ds(0, H)],
                        g_sh.at[pl.ds(s * RPT, RPT)])
        zv = jnp.zeros((16,), jnp.float32)

        def zbody(i, _):
            r = i // (H // 16)
            k = i % (H // 16)
            rows[0, 0, r, pl.ds(k * 16, 16)] = zv
            return 0

        lax.fori_loop(0, B * (H // 16), zbody, 0)
        nfull = RPT // B
        for q in range(nfull):
            pltpu.sync_copy(rows.at[0, 0],
                            acc.at[pl.ds(s * RPT + q * B, B)])
        rem = RPT - nfull * B
        if rem:
            pltpu.sync_copy(rows.at[0, 0].at[pl.ds(0, rem)],
                            acc.at[pl.ds(s * RPT + nfull * B, rem)])
        pltpu.sync_copy(src2d.at[pl.ds(w * PG, PG)], src_buf)
        pltpu.sync_copy(dst2d.at[pl.ds(w * PG, PG)], dst_buf)
        plsc.subcore_barrier()

        def fire_g(ch, p):
            for b in range(K):
                pltpu.async_copy(g_sh.at[src_buf.at[ch * K + b]],
                                 rows.at[p, b], gsem.at[p])

        def drain_g_fire_s(ch, p):
            for b in range(K):
                pltpu.make_async_copy(g_sh.at[src_buf.at[ch * K + b]],
                                      rows.at[p, b], gsem.at[p]).wait()
                pltpu.async_copy(rows.at[p, b],
                                 acc.at[dst_buf.at[ch * K + b]],
                                 ssem.at[p], add=True)

        def drain_s(ch, p):
            for b in range(K):
                pltpu.make_async_copy(rows.at[p, b],
                                      acc.at[dst_buf.at[ch * K + b]],
                                      ssem.at[p]).wait()

        fire_g(0, 0)
        fire_g(1, 1)
        drain_g_fire_s(0, 0)

        def body(ch, _):
            p = lax.rem(ch, 2)
            drain_s(ch - 2, p)
            fire_g(ch, p)
            drain_g_fire_s(ch - 1, 1 - p)
            return 0

        lax.fori_loop(2, CH, body, 0)
        drain_s(CH - 2, CH % 2)
        drain_g_fire_s(CH - 1, (CH - 1) % 2)
        drain_s(CH - 1, (CH - 1) % 2)
        plsc.subcore_barrier()

        pltpu.sync_copy(acc.at[pl.ds(s * RPT, RPT)],
                        out.at[c].at[pl.ds(s * RPT, RPT), pl.ds(0, H)])

    return scatter_kernel


def _tc_xw_body(x, w0, u_out):
    u_out[...] = jnp.dot(x[...], w0[...], preferred_element_type=jnp.float32)


def _make_tc0_body(N, NP):
    def _tc0_body(d0, d1, u, dinv_out, g0_out):
        H = u.shape[1]
        deg = d0[...] + d1[...] + 1.0
        dinv = lax.rsqrt(deg)
        dinv_out[...] = dinv
        g0_out[pl.ds(0, N), pl.ds(0, H)] = u[...] * dinv[:N, None]
        g0_out[pl.ds(N, NP - N), pl.ds(0, H)] = jnp.zeros((NP - N, H),
                                                          jnp.float32)
    return _tc0_body


def _tc_first_body(sp, s10, s11, g, dinv, b, w, c_out, gn_out):
    H = w.shape[0]
    dv = dinv[...]
    c_out[...] = dv * (s10[...] + s11[...] + dv)
    h = jnp.maximum(dv[:, None] * (sp[0, :, :H] + sp[1, :, :H] + g[:, :H])
                    + b[...][None, :], 0.0)
    gn_out[:, :H] = jnp.dot(h, w[...],
                            preferred_element_type=jnp.float32) * dv[:, None]


def _tc_mid_body(sp, g, dinv, b, w, gn_out):
    H = w.shape[0]
    dv = dinv[...]
    h = jnp.maximum(dv[:, None] * (sp[0, :, :H] + sp[1, :, :H] + g[:, :H])
                    + b[...][None, :], 0.0)
    gn_out[:, :H] = jnp.dot(h, w[...],
                            preferred_element_type=jnp.float32) * dv[:, None]


def _make_tc_final_body(N):
    def _tc_final_body(sp, g, dinv, b3, cvec, w4, b4, wlin, blin, out):
        H = w4.shape[0]
        dv = dinv[pl.ds(0, N)]
        h4 = jnp.maximum(
            dv[:, None] * (sp[0, :N, :H] + sp[1, :N, :H]
                           + g[:N, :H]) + b3[...][None, :], 0.0)
        r = jnp.sum(cvec[pl.ds(0, N)][:, None] * h4, axis=0,
                    keepdims=True) / N
        pooled = jnp.dot(r, w4[...], preferred_element_type=jnp.float32) \
            + b4[...][None, :]
        out[...] = jnp.dot(pooled, wlin[...],
                           preferred_element_type=jnp.float32) \
            + blin[...][None, :]
    return _tc_final_body


def kernel(x, edge_index, W0, b0, W1, b1, W2, b2, W3, b3, W4, b4, Wlin, blin):
    N, DIN = x.shape
    H = W0.shape[1]
    C = Wlin.shape[1]
    E = edge_index.shape[1]

    GALIGN = NW * 8
    NG = -(-E // B)
    NG = -(-NG // GALIGN) * GALIGN
    PG = NG // NW
    EP = NG * B
    NP = -(-(N + 1) // (NS * 8)) * (NS * 8)

    pad = jnp.full((2, EP - E), N, jnp.int32)
    ei = jnp.concatenate([edge_index, pad], axis=1)
    src2d = ei[0].reshape(NG, B)
    dst2d = ei[1].reshape(NG, B)
    zeros1 = jnp.zeros((NP,), jnp.float32)
    ones = jnp.ones((B,), jnp.float32)

    deg_k = _make_deg_kernel(NP, PG)
    scat0_k = _make_scatter0_kernel(NP, H, PG)
    scat_k = _make_scatter_kernel(NP, H, PG)

    fp32 = jnp.float32
    tc_xw = pl.pallas_call(_tc_xw_body,
                           out_shape=jax.ShapeDtypeStruct((N, H), fp32))
    tc0 = pl.pallas_call(_make_tc0_body(N, NP), out_shape=(
        jax.ShapeDtypeStruct((NP,), fp32),
        jax.ShapeDtypeStruct((NP, 128), fp32)))
    tc_first = pl.pallas_call(_tc_first_body, out_shape=(
        jax.ShapeDtypeStruct((NP,), fp32),
        jax.ShapeDtypeStruct((NP, 128), fp32)))
    tc_mid = pl.pallas_call(_tc_mid_body, out_shape=jax.ShapeDtypeStruct(
        (NP, 128), fp32))
    tc_final = pl.pallas_call(_make_tc_final_body(N),
                              out_shape=jax.ShapeDtypeStruct((1, C), fp32))

    u0 = tc_xw(x, W0)
    d0, d1 = deg_k(dst2d, zeros1, ones)
    dinv, g0 = tc0(d0, d1, u0)
    sp0, s10, s11 = scat0_k(g0, src2d, dst2d, dinv, zeros1)
    c, g1 = tc_first(sp0, s10, s11, g0, dinv, b0, W1)
    sp1 = scat_k(g1, src2d, dst2d)
    g2 = tc_mid(sp1, g1, dinv, b1, W2)
    sp2 = scat_k(g2, src2d, dst2d)
    g3 = tc_mid(sp2, g2, dinv, b2, W3)
    sp3 = scat_k(g3, src2d, dst2d)
    return tc_final(sp3, g3, dinv, b3, c, W4, b4, Wlin, blin)


# speedup vs baseline: 1.0059x; 1.0059x over previous
import functools

import jax
import jax.numpy as jnp
from jax import lax
from jax.experimental import pallas as pl
from jax.experimental.pallas import tpu as pltpu
from jax.experimental.pallas import tpu_sc as plsc

NC = 2
NS = 16
NW = NC * NS
B = 128
K = 4


def _worker_id():
    return lax.axis_index("s") * NC + lax.axis_index("c")


def _sc_mesh():
    return plsc.VectorSubcoreMesh(core_axis_name="c", subcore_axis_name="s")


_SC_PARAMS = pltpu.CompilerParams(use_tc_tiling_on_sc=False)


def _make_deg_kernel(NP, PG):
    @functools.partial(
        pl.kernel,
        mesh=_sc_mesh(),
        compiler_params=_SC_PARAMS,
        out_type=(jax.ShapeDtypeStruct((NP,), jnp.float32),
                  jax.ShapeDtypeStruct((NP,), jnp.float32)),
        scratch_types=[
            pltpu.VMEM((PG, B), jnp.int32),
            pltpu.VMEM((B,), jnp.float32),
            pltpu.VMEM_SHARED((NP,), jnp.float32),
            pltpu.SemaphoreType.DMA,
        ],
    )
    def deg_kernel(dst2d, zeros1, ones, out0, out1, dst_buf, ones_buf, acc,
                   ssem):
        c = lax.axis_index("c")
        s = lax.axis_index("s")
        w = _worker_id()

        @pl.when(s == 0)
        def _():
            pltpu.sync_copy(zeros1, acc)
        pltpu.sync_copy(ones, ones_buf)
        pltpu.sync_copy(dst2d.at[pl.ds(w * PG, PG)], dst_buf)
        plsc.subcore_barrier()

        def fire(j, _):
            pltpu.async_copy(ones_buf, acc.at[dst_buf.at[j]], ssem, add=True)
            return 0

        lax.fori_loop(0, PG, fire, 0)

        def drain(j, _):
            pltpu.make_async_copy(ones_buf, acc.at[dst_buf.at[j]], ssem).wait()
            return 0

        lax.fori_loop(0, PG, drain, 0)
        plsc.subcore_barrier()

        @pl.when((s == 0) & (c == 0))
        def _():
            pltpu.sync_copy(acc, out0)

        @pl.when((s == 0) & (c == 1))
        def _():
            pltpu.sync_copy(acc, out1)

    return deg_kernel


def _make_s1_kernel(NP, PG):
    @functools.partial(
        pl.kernel,
        mesh=_sc_mesh(),
        compiler_params=_SC_PARAMS,
        out_type=(jax.ShapeDtypeStruct((NP,), jnp.float32),
                  jax.ShapeDtypeStruct((NP,), jnp.float32)),
        scratch_types=[
            pltpu.VMEM((PG, B), jnp.int32),
            pltpu.VMEM((PG, B), jnp.int32),
            pltpu.VMEM((PG, B), jnp.float32),
            pltpu.VMEM_SHARED((NP,), jnp.float32),
            pltpu.VMEM_SHARED((NP,), jnp.float32),
            pltpu.SemaphoreType.DMA,
            pltpu.SemaphoreType.DMA,
        ],
    )
    def s1_kernel(src2d, dst2d, dinv, zeros1, out0, out1, src_buf, dst_buf,
                  vals, acc, dinv_sh, gsem, ssem):
        c = lax.axis_index("c")
        s = lax.axis_index("s")
        w = _worker_id()
        RP1 = NP // NS

        @pl.when(s == 0)
        def _():
            pltpu.sync_copy(zeros1, acc)
        pltpu.sync_copy(dinv.at[pl.ds(s * RP1, RP1)],
                        dinv_sh.at[pl.ds(s * RP1, RP1)])
        pltpu.sync_copy(src2d.at[pl.ds(w * PG, PG)], src_buf)
        pltpu.sync_copy(dst2d.at[pl.ds(w * PG, PG)], dst_buf)
        plsc.subcore_barrier()

        def fire_g(j, _):
            pltpu.async_copy(dinv_sh.at[dst_buf.at[j]], vals.at[j], gsem)
            return 0

        lax.fori_loop(0, PG, fire_g, 0)

        def drain_g(j, _):
            pltpu.make_async_copy(dinv_sh.at[dst_buf.at[j]], vals.at[j],
                                  gsem).wait()
            return 0

        lax.fori_loop(0, PG, drain_g, 0)

        def fire_s(j, _):
            pltpu.async_copy(vals.at[j], acc.at[src_buf.at[j]], ssem,
                             add=True)
            return 0

        lax.fori_loop(0, PG, fire_s, 0)

        def drain_s(j, _):
            pltpu.make_async_copy(vals.at[j], acc.at[src_buf.at[j]],
                                  ssem).wait()
            return 0

        lax.fori_loop(0, PG, drain_s, 0)
        plsc.subcore_barrier()

        @pl.when((s == 0) & (c == 0))
        def _():
            pltpu.sync_copy(acc, out0)

        @pl.when((s == 0) & (c == 1))
        def _():
            pltpu.sync_copy(acc, out1)

    return s1_kernel


def _make_scatter_kernel(NP, H, PG):
    RPT = NP // NS
    CH = PG // K
    assert PG % K == 0 and CH >= 4

    @functools.partial(
        pl.kernel,
        mesh=_sc_mesh(),
        compiler_params=_SC_PARAMS,
        out_type=jax.ShapeDtypeStruct((NC, NP, 128), jnp.float32),
        scratch_types=[
            pltpu.VMEM((PG, B), jnp.int32),
            pltpu.VMEM((PG, B), jnp.int32),
            pltpu.VMEM((2, K, B, H), jnp.float32),
            pltpu.VMEM_SHARED((NP, H), jnp.float32),
            pltpu.VMEM_SHARED((NP, H), jnp.float32),
            pltpu.SemaphoreType.DMA((2,)),
            pltpu.SemaphoreType.DMA((2,)),
        ],
    )
    def scatter_kernel(g, src2d, dst2d, out, src_buf, dst_buf, rows,
                       acc, g_sh, gsem, ssem):
        c = lax.axis_index("c")
        s = lax.axis_index("s")
        w = _worker_id()

        pltpu.sync_copy(g.at[pl.ds(s * RPT, RPT), pl.ds(0, H)],
                        g_sh.at[pl.ds(s * RPT, RPT)])
        zv = jnp.zeros((16,), jnp.float32)

        def zbody(i, _):
            r = i // (H // 16)
            k = i % (H // 16)
            rows[0, 0, r, pl.ds(k * 16, 16)] = zv
            return 0

        lax.fori_loop(0, B * (H // 16), zbody, 0)
        nfull = RPT // B
        for q in range(nfull):
            pltpu.sync_copy(rows.at[0, 0],
                            acc.at[pl.ds(s * RPT + q * B, B)])
        rem = RPT - nfull * B
        if rem:
            pltpu.sync_copy(rows.at[0, 0].at[pl.ds(0, rem)],
                            acc.at[pl.ds(s * RPT + nfull * B, rem)])
        pltpu.sync_copy(src2d.at[pl.ds(w * PG, PG)], src_buf)
        pltpu.sync_copy(dst2d.at[pl.ds(w * PG, PG)], dst_buf)
        plsc.subcore_barrier()

        def fire_g(ch, p):
            for b in range(K):
                pltpu.async_copy(g_sh.at[src_buf.at[ch * K + b]],
                                 rows.at[p, b], gsem.at[p])

        def drain_g_fire_s(ch, p):
            for b in range(K):
                pltpu.make_async_copy(g_sh.at[src_buf.at[ch * K + b]],
                                      rows.at[p, b], gsem.at[p]).wait()
                pltpu.async_copy(rows.at[p, b],
                                 acc.at[dst_buf.at[ch * K + b]],
                                 ssem.at[p], add=True)

        def drain_s(ch, p):
            for b in range(K):
                pltpu.make_async_copy(rows.at[p, b],
                                      acc.at[dst_buf.at[ch * K + b]],
                                      ssem.at[p]).wait()

        fire_g(0, 0)
        fire_g(1, 1)
        drain_g_fire_s(0, 0)

        def body(ch, _):
            p = lax.rem(ch, 2)
            drain_s(ch - 2, p)
            fire_g(ch, p)
            drain_g_fire_s(ch - 1, 1 - p)
            return 0

        lax.fori_loop(2, CH, body, 0)
        drain_s(CH - 2, CH % 2)
        drain_g_fire_s(CH - 1, (CH - 1) % 2)
        drain_s(CH - 1, (CH - 1) % 2)
        plsc.subcore_barrier()

        pltpu.sync_copy(acc.at[pl.ds(s * RPT, RPT)],
                        out.at[c].at[pl.ds(s * RPT, RPT), pl.ds(0, H)])

    return scatter_kernel


def _tc_xw_body(x, w0, u_out):
    u_out[...] = jnp.dot(x[...], w0[...], preferred_element_type=jnp.float32)


def _make_tc0_body(N, NP):
    def _tc0_body(d0, d1, u, dinv_out, g0_out):
        H = u.shape[1]
        deg = d0[...] + d1[...] + 1.0
        dinv = lax.rsqrt(deg)
        dinv_out[...] = dinv
        g0_out[pl.ds(0, N), pl.ds(0, H)] = u[...] * dinv[:N, None]
        g0_out[pl.ds(N, NP - N), pl.ds(0, H)] = jnp.zeros((NP - N, H),
                                                          jnp.float32)
    return _tc0_body


def _tc_first_body(sp, s10, s11, g, dinv, b, w, c_out, gn_out):
    H = w.shape[0]
    dv = dinv[...]
    c_out[...] = dv * (s10[...] + s11[...] + dv)
    h = jnp.maximum(dv[:, None] * (sp[0, :, :H] + sp[1, :, :H] + g[:, :H])
                    + b[...][None, :], 0.0)
    gn_out[:, :H] = jnp.dot(h, w[...],
                            preferred_element_type=jnp.float32) * dv[:, None]


def _tc_mid_body(sp, g, dinv, b, w, gn_out):
    H = w.shape[0]
    dv = dinv[...]
    h = jnp.maximum(dv[:, None] * (sp[0, :, :H] + sp[1, :, :H] + g[:, :H])
                    + b[...][None, :], 0.0)
    gn_out[:, :H] = jnp.dot(h, w[...],
                            preferred_element_type=jnp.float32) * dv[:, None]


def _make_tc_final_body(N):
    def _tc_final_body(sp, g, dinv, b3, cvec, w4, b4, wlin, blin, out):
        H = w4.shape[0]
        dv = dinv[pl.ds(0, N)]
        h4 = jnp.maximum(
            dv[:, None] * (sp[0, :N, :H] + sp[1, :N, :H]
                           + g[:N, :H]) + b3[...][None, :], 0.0)
        r = jnp.sum(cvec[pl.ds(0, N)][:, None] * h4, axis=0,
                    keepdims=True) / N
        pooled = jnp.dot(r, w4[...], preferred_element_type=jnp.float32) \
            + b4[...][None, :]
        out[...] = jnp.dot(pooled, wlin[...],
                           preferred_element_type=jnp.float32) \
            + blin[...][None, :]
    return _tc_final_body


def kernel(x, edge_index, W0, b0, W1, b1, W2, b2, W3, b3, W4, b4, Wlin, blin):
    N, DIN = x.shape
    H = W0.shape[1]
    C = Wlin.shape[1]
    E = edge_index.shape[1]

    GALIGN = NW * 8
    NG = -(-E // B)
    NG = -(-NG // GALIGN) * GALIGN
    PG = NG // NW
    EP = NG * B
    NP = -(-(N + 1) // (NS * 8)) * (NS * 8)

    pad = jnp.full((2, EP - E), N, jnp.int32)
    ei = jnp.concatenate([edge_index, pad], axis=1)
    src2d = ei[0].reshape(NG, B)
    dst2d = ei[1].reshape(NG, B)
    zeros1 = jnp.zeros((NP,), jnp.float32)
    ones = jnp.ones((B,), jnp.float32)

    deg_k = _make_deg_kernel(NP, PG)
    s1_k = _make_s1_kernel(NP, PG)
    scat_k = _make_scatter_kernel(NP, H, PG)

    fp32 = jnp.float32
    tc_xw = pl.pallas_call(_tc_xw_body,
                           out_shape=jax.ShapeDtypeStruct((N, H), fp32))
    tc0 = pl.pallas_call(_make_tc0_body(N, NP), out_shape=(
        jax.ShapeDtypeStruct((NP,), fp32),
        jax.ShapeDtypeStruct((NP, 128), fp32)))
    tc_first = pl.pallas_call(_tc_first_body, out_shape=(
        jax.ShapeDtypeStruct((NP,), fp32),
        jax.ShapeDtypeStruct((NP, 128), fp32)))
    tc_mid = pl.pallas_call(_tc_mid_body, out_shape=jax.ShapeDtypeStruct(
        (NP, 128), fp32))
    tc_final = pl.pallas_call(_make_tc_final_body(N),
                              out_shape=jax.ShapeDtypeStruct((1, C), fp32))

    u0 = tc_xw(x, W0)
    d0, d1 = deg_k(dst2d, zeros1, ones)
    dinv, g0 = tc0(d0, d1, u0)
    s10, s11 = s1_k(src2d, dst2d, dinv, zeros1)
    sp0 = scat_k(g0, src2d, dst2d)
    c, g1 = tc_first(sp0, s10, s11, g0, dinv, b0, W1)
    sp1 = scat_k(g1, src2d, dst2d)
    g2 = tc_mid(sp1, g1, dinv, b1, W2)
    sp2 = scat_k(g2, src2d, dst2d)
    g3 = tc_mid(sp2, g2, dinv, b2, W3)
    sp3 = scat_k(g3, src2d, dst2d)
    return tc_final(sp3, g3, dinv, b3, c, W4, b4, Wlin, blin)


# speedup vs baseline: 1.0125x; 1.0066x over previous
import functools

import jax
import jax.numpy as jnp
from jax import lax
from jax.experimental import pallas as pl
from jax.experimental.pallas import tpu as pltpu
from jax.experimental.pallas import tpu_sc as plsc

NC = 2
NS = 16
NW = NC * NS
B = 128
K = 4


def _worker_id():
    return lax.axis_index("s") * NC + lax.axis_index("c")


def _sc_mesh():
    return plsc.VectorSubcoreMesh(core_axis_name="c", subcore_axis_name="s")


_SC_PARAMS = pltpu.CompilerParams(use_tc_tiling_on_sc=False)


def _make_deg_kernel(NP, PG):
    @functools.partial(
        pl.kernel,
        mesh=_sc_mesh(),
        compiler_params=_SC_PARAMS,
        out_type=(jax.ShapeDtypeStruct((NP,), jnp.float32),
                  jax.ShapeDtypeStruct((NP,), jnp.float32)),
        scratch_types=[
            pltpu.VMEM((PG, B), jnp.int32),
            pltpu.VMEM((B,), jnp.float32),
            pltpu.VMEM_SHARED((NP,), jnp.float32),
            pltpu.SemaphoreType.DMA,
        ],
    )
    def deg_kernel(dst2d, zeros1, ones, out0, out1, dst_buf, ones_buf, acc,
                   ssem):
        c = lax.axis_index("c")
        s = lax.axis_index("s")
        w = _worker_id()

        @pl.when(s == 0)
        def _():
            pltpu.sync_copy(zeros1, acc)
        pltpu.sync_copy(ones, ones_buf)
        pltpu.sync_copy(dst2d.at[pl.ds(w * PG, PG)], dst_buf)
        plsc.subcore_barrier()

        def fire(j, _):
            pltpu.async_copy(ones_buf, acc.at[dst_buf.at[j]], ssem, add=True)
            return 0

        lax.fori_loop(0, PG, fire, 0)

        def drain(j, _):
            pltpu.make_async_copy(ones_buf, acc.at[dst_buf.at[j]], ssem).wait()
            return 0

        lax.fori_loop(0, PG, drain, 0)
        plsc.subcore_barrier()

        @pl.when((s == 0) & (c == 0))
        def _():
            pltpu.sync_copy(acc, out0)

        @pl.when((s == 0) & (c == 1))
        def _():
            pltpu.sync_copy(acc, out1)

    return deg_kernel


def _make_s1_kernel(NP, PG):
    @functools.partial(
        pl.kernel,
        mesh=_sc_mesh(),
        compiler_params=_SC_PARAMS,
        out_type=(jax.ShapeDtypeStruct((NP,), jnp.float32),
                  jax.ShapeDtypeStruct((NP,), jnp.float32)),
        scratch_types=[
            pltpu.VMEM((PG, B), jnp.int32),
            pltpu.VMEM((PG, B), jnp.int32),
            pltpu.VMEM((PG, B), jnp.float32),
            pltpu.VMEM_SHARED((NP,), jnp.float32),
            pltpu.VMEM_SHARED((NP,), jnp.float32),
            pltpu.SemaphoreType.DMA,
            pltpu.SemaphoreType.DMA,
        ],
    )
    def s1_kernel(src2d, dst2d, dinv, zeros1, out0, out1, src_buf, dst_buf,
                  vals, acc, dinv_sh, gsem, ssem):
        c = lax.axis_index("c")
        s = lax.axis_index("s")
        w = _worker_id()
        RP1 = NP // NS

        @pl.when(s == 0)
        def _():
            pltpu.sync_copy(zeros1, acc)
        pltpu.sync_copy(dinv.at[pl.ds(s * RP1, RP1)],
                        dinv_sh.at[pl.ds(s * RP1, RP1)])
        pltpu.sync_copy(src2d.at[pl.ds(w * PG, PG)], src_buf)
        pltpu.sync_copy(dst2d.at[pl.ds(w * PG, PG)], dst_buf)
        plsc.subcore_barrier()

        def fire_g(j, _):
            pltpu.async_copy(dinv_sh.at[dst_buf.at[j]], vals.at[j], gsem)
            return 0

        lax.fori_loop(0, PG, fire_g, 0)

        def drain_g(j, _):
            pltpu.make_async_copy(dinv_sh.at[dst_buf.at[j]], vals.at[j],
                                  gsem).wait()
            return 0

        lax.fori_loop(0, PG, drain_g, 0)

        def fire_s(j, _):
            pltpu.async_copy(vals.at[j], acc.at[src_buf.at[j]], ssem,
                             add=True)
            return 0

        lax.fori_loop(0, PG, fire_s, 0)

        def drain_s(j, _):
            pltpu.make_async_copy(vals.at[j], acc.at[src_buf.at[j]],
                                  ssem).wait()
            return 0

        lax.fori_loop(0, PG, drain_s, 0)
        plsc.subcore_barrier()

        @pl.when((s == 0) & (c == 0))
        def _():
            pltpu.sync_copy(acc, out0)

        @pl.when((s == 0) & (c == 1))
        def _():
            pltpu.sync_copy(acc, out1)

    return s1_kernel


def _make_scatter_kernel(NP, H, PG):
    RPT = NP // NS
    CH = PG // K
    assert PG % K == 0 and CH >= 4

    @functools.partial(
        pl.kernel,
        mesh=_sc_mesh(),
        compiler_params=_SC_PARAMS,
        out_type=jax.ShapeDtypeStruct((NC, NP, 128), jnp.float32),
        scratch_types=[
            pltpu.VMEM((PG, B), jnp.int32),
            pltpu.VMEM((PG, B), jnp.int32),
            pltpu.VMEM((2, K, B, H), jnp.float32),
            pltpu.VMEM_SHARED((NP, H), jnp.float32),
            pltpu.VMEM_SHARED((NP, H), jnp.float32),
            pltpu.SemaphoreType.DMA((2,)),
            pltpu.SemaphoreType.DMA((2,)),
        ],
    )
    def scatter_kernel(g, src2d, dst2d, out, src_buf, dst_buf, rows,
                       acc, g_sh, gsem, ssem):
        c = lax.axis_index("c")
        s = lax.axis_index("s")
        w = _worker_id()

        pltpu.sync_copy(g.at[pl.ds(s * RPT, RPT), pl.ds(0, H)],
                        g_sh.at[pl.ds(s * RPT, RPT)])
        zv = jnp.zeros((16,), jnp.float32)

        def zbody(i, _):
            r = i // (H // 16)
            k = i % (H // 16)
            rows[0, 0, r, pl.ds(k * 16, 16)] = zv
            return 0

        lax.fori_loop(0, B * (H // 16), zbody, 0)
        nfull = RPT // B
        for q in range(nfull):
            pltpu.sync_copy(rows.at[0, 0],
                            acc.at[pl.ds(s * RPT + q * B, B)])
        rem = RPT - nfull * B
        if rem:
            pltpu.sync_copy(rows.at[0, 0].at[pl.ds(0, rem)],
                            acc.at[pl.ds(s * RPT + nfull * B, rem)])
        pltpu.sync_copy(src2d.at[pl.ds(w * PG, PG)], src_buf)
        pltpu.sync_copy(dst2d.at[pl.ds(w * PG, PG)], dst_buf)
        plsc.subcore_barrier()

        def fire_g(ch, p):
            for b in range(K):
                pltpu.async_copy(g_sh.at[src_buf.at[ch * K + b]],
                                 rows.at[p, b], gsem.at[p])

        def drain_g_fire_s(ch, p):
            for b in range(K):
                pltpu.make_async_copy(g_sh.at[src_buf.at[ch * K + b]],
                                      rows.at[p, b], gsem.at[p]).wait()
                pltpu.async_copy(rows.at[p, b],
                                 acc.at[dst_buf.at[ch * K + b]],
                                 ssem.at[p], add=True)

        def drain_s(ch, p):
            for b in range(K):
                pltpu.make_async_copy(rows.at[p, b],
                                      acc.at[dst_buf.at[ch * K + b]],
                                      ssem.at[p]).wait()

        fire_g(0, 0)
        fire_g(1, 1)
        drain_g_fire_s(0, 0)

        def body(ch, _):
            p = lax.rem(ch, 2)
            drain_s(ch - 2, p)
            fire_g(ch, p)
            drain_g_fire_s(ch - 1, 1 - p)
            return 0

        lax.fori_loop(2, CH, body, 0)
        drain_s(CH - 2, CH % 2)
        drain_g_fire_s(CH - 1, (CH - 1) % 2)
        drain_s(CH - 1, (CH - 1) % 2)
        plsc.subcore_barrier()

        pltpu.sync_copy(acc.at[pl.ds(s * RPT, RPT)],
                        out.at[c].at[pl.ds(s * RPT, RPT), pl.ds(0, H)])

    return scatter_kernel


def _make_tc0_body(N, NP):
    def _tc0_body(d0, d1, x, w0, dinv_out, g0_out):
        H = w0.shape[1]
        deg = d0[...] + d1[...] + 1.0
        dinv = lax.rsqrt(deg)
        dinv_out[...] = dinv
        g0 = jnp.dot(x[...], w0[...],
                     preferred_element_type=jnp.float32) * dinv[:N, None]
        g0_out[pl.ds(0, N), pl.ds(0, H)] = g0
        g0_out[pl.ds(N, NP - N), pl.ds(0, H)] = jnp.zeros((NP - N, H),
                                                          jnp.float32)
    return _tc0_body


def _tc_first_body(sp, s10, s11, g, dinv, b, w, c_out, gn_out):
    H = w.shape[0]
    dv = dinv[...]
    c_out[...] = dv * (s10[...] + s11[...] + dv)
    h = jnp.maximum(dv[:, None] * (sp[0, :, :H] + sp[1, :, :H] + g[:, :H])
                    + b[...][None, :], 0.0)
    gn_out[:, :H] = jnp.dot(h, w[...],
                            preferred_element_type=jnp.float32) * dv[:, None]


def _tc_mid_body(sp, g, dinv, b, w, gn_out):
    H = w.shape[0]
    dv = dinv[...]
    h = jnp.maximum(dv[:, None] * (sp[0, :, :H] + sp[1, :, :H] + g[:, :H])
                    + b[...][None, :], 0.0)
    gn_out[:, :H] = jnp.dot(h, w[...],
                            preferred_element_type=jnp.float32) * dv[:, None]


def _make_tc_final_body(N):
    def _tc_final_body(sp, g, dinv, b3, cvec, w4, b4, wlin, blin, out):
        H = w4.shape[0]
        dv = dinv[pl.ds(0, N)]
        h4 = jnp.maximum(
            dv[:, None] * (sp[0, :N, :H] + sp[1, :N, :H]
                           + g[:N, :H]) + b3[...][None, :], 0.0)
        r = jnp.sum(cvec[pl.ds(0, N)][:, None] * h4, axis=0,
                    keepdims=True) / N
        pooled = jnp.dot(r, w4[...], preferred_element_type=jnp.float32) \
            + b4[...][None, :]
        out[...] = jnp.dot(pooled, wlin[...],
                           preferred_element_type=jnp.float32) \
            + blin[...][None, :]
    return _tc_final_body


def kernel(x, edge_index, W0, b0, W1, b1, W2, b2, W3, b3, W4, b4, Wlin, blin):
    N, DIN = x.shape
    H = W0.shape[1]
    C = Wlin.shape[1]
    E = edge_index.shape[1]

    GALIGN = NW * 8
    NG = -(-E // B)
    NG = -(-NG // GALIGN) * GALIGN
    PG = NG // NW
    EP = NG * B
    NP = -(-(N + 1) // (NS * 8)) * (NS * 8)

    pad = jnp.full((2, EP - E), N, jnp.int32)
    ei = jnp.concatenate([edge_index, pad], axis=1)
    src2d = ei[0].reshape(NG, B)
    dst2d = ei[1].reshape(NG, B)
    zeros1 = jnp.zeros((NP,), jnp.float32)
    ones = jnp.ones((B,), jnp.float32)

    deg_k = _make_deg_kernel(NP, PG)
    s1_k = _make_s1_kernel(NP, PG)
    scat_k = _make_scatter_kernel(NP, H, PG)

    fp32 = jnp.float32
    tc0 = pl.pallas_call(_make_tc0_body(N, NP), out_shape=(
        jax.ShapeDtypeStruct((NP,), fp32),
        jax.ShapeDtypeStruct((NP, 128), fp32)))
    tc_first = pl.pallas_call(_tc_first_body, out_shape=(
        jax.ShapeDtypeStruct((NP,), fp32),
        jax.ShapeDtypeStruct((NP, 128), fp32)))
    tc_mid = pl.pallas_call(_tc_mid_body, out_shape=jax.ShapeDtypeStruct(
        (NP, 128), fp32))
    tc_final = pl.pallas_call(_make_tc_final_body(N),
                              out_shape=jax.ShapeDtypeStruct((1, C), fp32))

    d0, d1 = deg_k(dst2d, zeros1, ones)
    dinv, g0 = tc0(d0, d1, x, W0)
    s10, s11 = s1_k(src2d, dst2d, dinv, zeros1)
    sp0 = scat_k(g0, src2d, dst2d)
    c, g1 = tc_first(sp0, s10, s11, g0, dinv, b0, W1)
    sp1 = scat_k(g1, src2d, dst2d)
    g2 = tc_mid(sp1, g1, dinv, b1, W2)
    sp2 = scat_k(g2, src2d, dst2d)
    g3 = tc_mid(sp2, g2, dinv, b2, W3)
    sp3 = scat_k(g3, src2d, dst2d)
    return tc_final(sp3, g3, dinv, b3, c, W4, b4, Wlin, blin)
